# baseline (speedup 1.0000x reference)
import functools

import jax
import jax.numpy as jnp
from jax import lax
from jax.experimental import pallas as pl
from jax.experimental.pallas import tpu as pltpu
from jax.experimental.pallas import tpu_sc as plsc

N = 4096
D_X = 128
D_Y = 2
K = 2500
K_PAD = 2560
KT = K_PAD // 128
N_BLK = 512
N_GRID = N // N_BLK
A_PAD = 2.0e18
U_COLS = 128


def _prep_body(b_ref, ap_ref, u_ref, w_ref, a_ref, t_ref):
    j = pl.program_id(0)
    lane = lax.broadcasted_iota(jnp.int32, (1, 128), 1)
    valid = (j * 128 + lane) < K
    for d in range(D_Y):
        w_ref[d] = jnp.where(valid, jnp.transpose(b_ref[:, d, :], (1, 0)), 0.0)
        a_ref[d] = jnp.where(valid, jnp.transpose(ap_ref[:, d:d + 1], (1, 0)),
                             A_PAD)
    li = lax.broadcasted_iota(jnp.int32, (128, U_COLS), 1)
    t_ref[...] = jnp.where(li == 0, u_ref[:, 0:1],
                           jnp.where(li == 1, u_ref[:, 1:2], 0.0))


def _prep(B, A, u_grid):
    return pl.pallas_call(
        _prep_body,
        grid=(KT,),
        in_specs=[
            pl.BlockSpec((128, D_Y, D_X), lambda j: (j, 0, 0)),
            pl.BlockSpec((128, D_Y), lambda j: (j, 0)),
            pl.BlockSpec((128, D_Y), lambda j: (j, 0)),
        ],
        out_specs=[
            pl.BlockSpec((D_Y, D_X, 128), lambda j: (0, 0, j)),
            pl.BlockSpec((D_Y, 1, 128), lambda j: (0, 0, j)),
            pl.BlockSpec((128, U_COLS), lambda j: (j, 0)),
        ],
        out_shape=[
            jax.ShapeDtypeStruct((D_Y, D_X, K_PAD), jnp.float32),
            jax.ShapeDtypeStruct((D_Y, 1, K_PAD), jnp.float32),
            jax.ShapeDtypeStruct((K_PAD, U_COLS), jnp.float32),
        ],
    )(B, A, u_grid)


def _dist_argmin_body(x_ref, w_ref, a_ref, y_ref, idx_ref):
    x = x_ref[...]
    s0 = jnp.dot(x, w_ref[0], preferred_element_type=jnp.float32) + a_ref[0]
    s1 = jnp.dot(x, w_ref[1], preferred_element_type=jnp.float32) + a_ref[1]
    d0 = s0 - y_ref[:, 0:1]
    d1 = s1 - y_ref[:, 1:2]
    d2 = d0 * d0 + d1 * d1
    idx = jnp.argmin(d2, axis=1).astype(jnp.int32)
    idx_ref[0] = idx.reshape(N_BLK // 128, 128)


def _argmin_indices(x, w, a, y):
    return pl.pallas_call(
        _dist_argmin_body,
        grid=(N_GRID,),
        in_specs=[
            pl.BlockSpec((N_BLK, D_X), lambda i: (i, 0)),
            pl.BlockSpec((D_Y, D_X, K_PAD), lambda i: (0, 0, 0)),
            pl.BlockSpec((D_Y, 1, K_PAD), lambda i: (0, 0, 0)),
            pl.BlockSpec((N_BLK, D_Y), lambda i: (i, 0)),
        ],
        out_specs=pl.BlockSpec((1, N_BLK // 128, 128), lambda i: (i, 0, 0)),
        out_shape=jax.ShapeDtypeStruct((N_GRID, N_BLK // 128, 128), jnp.int32),
    )(x, w, a, y)


def _sc_gather(table, idx):
    info = plsc.get_sparse_core_info()
    nc, ns = info.num_cores, info.num_subcores
    nw = nc * ns
    b_per_w = N // nw
    rows_per_blk = N_BLK // 128
    mesh = plsc.VectorSubcoreMesh(core_axis_name="c", subcore_axis_name="s")

    @functools.partial(
        pl.kernel,
        mesh=mesh,
        out_type=jax.ShapeDtypeStruct((N, U_COLS), jnp.float32),
        scratch_types=[
            pltpu.VMEM((b_per_w,), jnp.int32),
            pltpu.VMEM((b_per_w, U_COLS), jnp.float32),
            pltpu.SemaphoreType.DMA,
        ],
    )
    def gather_kernel(table_hbm, idx_hbm, out_hbm, idx_v, rows_v, sem):
        wid = lax.axis_index("s") * nc + lax.axis_index("c")
        pltpu.sync_copy(idx_hbm.at[wid // rows_per_blk, wid % rows_per_blk],
                        idx_v)
        pltpu.async_copy(table_hbm.at[idx_v], rows_v, sem).wait()
        pltpu.sync_copy(rows_v, out_hbm.at[pl.ds(wid * b_per_w, b_per_w)])

    return gather_kernel(table, idx)


def kernel(y, x, B, A, u_grid):
    w, a, table = _prep(B, A, u_grid)
    idx = _argmin_indices(x, w, a, y)
    return _sc_gather(table, idx)[:, :D_Y]


# speedup vs baseline: 1.2041x; 1.2041x over previous
import functools

import jax
import jax.numpy as jnp
from jax import lax
from jax.experimental import pallas as pl
from jax.experimental.pallas import tpu as pltpu
from jax.experimental.pallas import tpu_sc as plsc

N = 4096
D_X = 128
D_Y = 2
K = 2500
K_PAD = 2560
KT = K_PAD // 128
N_BLK = 512
N_GRID = N // N_BLK
A_PAD = 2.0e18
U_COLS = 128


def _prep_body(b_ref, ap_ref, u_ref, w_ref, a_ref, t_ref):
    j = pl.program_id(0)
    lane = lax.broadcasted_iota(jnp.int32, (1, 128), 1)
    valid = (j * 128 + lane) < K
    for d in range(D_Y):
        w_ref[d] = jnp.where(valid, jnp.transpose(b_ref[:, d, :], (1, 0)), 0.0)
        a_ref[d] = jnp.where(valid, jnp.transpose(ap_ref[:, d:d + 1], (1, 0)),
                             A_PAD)
    li = lax.broadcasted_iota(jnp.int32, (128, U_COLS), 1)
    t_ref[...] = jnp.where(li == 0, u_ref[:, 0:1],
                           jnp.where(li == 1, u_ref[:, 1:2], 0.0))


def _prep(B, A, u_grid):
    return pl.pallas_call(
        _prep_body,
        grid=(KT,),
        in_specs=[
            pl.BlockSpec((128, D_Y, D_X), lambda j: (j, 0, 0)),
            pl.BlockSpec((128, D_Y), lambda j: (j, 0)),
            pl.BlockSpec((128, D_Y), lambda j: (j, 0)),
        ],
        out_specs=[
            pl.BlockSpec((D_Y, D_X, 128), lambda j: (0, 0, j)),
            pl.BlockSpec((D_Y, 1, 128), lambda j: (0, 0, j)),
            pl.BlockSpec((128, U_COLS), lambda j: (j, 0)),
        ],
        out_shape=[
            jax.ShapeDtypeStruct((D_Y, D_X, K_PAD), jnp.float32),
            jax.ShapeDtypeStruct((D_Y, 1, K_PAD), jnp.float32),
            jax.ShapeDtypeStruct((K_PAD, U_COLS), jnp.float32),
        ],
    )(B, A, u_grid)


def _dist_argmin_body(x_ref, w_ref, a_ref, y_ref, idx_ref):
    x = x_ref[...]
    s0 = jnp.dot(x, w_ref[0], preferred_element_type=jnp.float32) + a_ref[0]
    s1 = jnp.dot(x, w_ref[1], preferred_element_type=jnp.float32) + a_ref[1]
    d0 = s0 - y_ref[:, 0:1]
    d1 = s1 - y_ref[:, 1:2]
    d2 = d0 * d0 + d1 * d1
    idx = jnp.argmin(d2, axis=1).astype(jnp.int32)
    idx_ref[0] = idx.reshape(N_BLK // 128, 128)


def _argmin_indices(x, w, a, y):
    return pl.pallas_call(
        _dist_argmin_body,
        grid=(N_GRID,),
        in_specs=[
            pl.BlockSpec((N_BLK, D_X), lambda i: (i, 0)),
            pl.BlockSpec((D_Y, D_X, K_PAD), lambda i: (0, 0, 0)),
            pl.BlockSpec((D_Y, 1, K_PAD), lambda i: (0, 0, 0)),
            pl.BlockSpec((N_BLK, D_Y), lambda i: (i, 0)),
        ],
        out_specs=pl.BlockSpec((1, N_BLK // 128, 128), lambda i: (i, 0, 0)),
        out_shape=jax.ShapeDtypeStruct((N_GRID, N_BLK // 128, 128), jnp.int32),
    )(x, w, a, y)


def _sc_gather(table, idx):
    info = plsc.get_sparse_core_info()
    nc, ns = info.num_cores, info.num_subcores
    nw = nc * ns
    b_per_w = N // nw
    rows_per_blk = N_BLK // 128
    mesh = plsc.VectorSubcoreMesh(core_axis_name="c", subcore_axis_name="s")

    @functools.partial(
        pl.kernel,
        mesh=mesh,
        out_type=jax.ShapeDtypeStruct((N, U_COLS), jnp.float32),
        scratch_types=[
            pltpu.VMEM((b_per_w,), jnp.int32),
            pltpu.VMEM((b_per_w, U_COLS), jnp.float32),
            pltpu.SemaphoreType.DMA,
        ],
    )
    def gather_kernel(table_hbm, idx_hbm, out_hbm, idx_v, rows_v, sem):
        wid = lax.axis_index("s") * nc + lax.axis_index("c")
        pltpu.sync_copy(idx_hbm.at[wid // rows_per_blk, wid % rows_per_blk],
                        idx_v)
        pltpu.async_copy(table_hbm.at[idx_v], rows_v, sem).wait()
        pltpu.sync_copy(rows_v, out_hbm.at[pl.ds(wid * b_per_w, b_per_w)])

    return gather_kernel(table, idx)


def kernel(y, x, B, A, u_grid):
    w = jnp.pad(jnp.transpose(B, (1, 2, 0)),
                ((0, 0), (0, 0), (0, K_PAD - K)))
    a = jnp.pad(A.T[:, None, :], ((0, 0), (0, 0), (0, K_PAD - K)),
                constant_values=A_PAD)
    table = jnp.pad(u_grid, ((0, K_PAD - K), (0, U_COLS - D_Y)))
    idx = _argmin_indices(x, w, a, y)
    return _sc_gather(table, idx)[:, :D_Y]
